# baseline (speedup 1.0000x reference)
import functools

import jax
import jax.numpy as jnp
from jax import lax
from jax.experimental import pallas as pl
from jax.experimental.pallas import tpu as pltpu
from jax.experimental.pallas import tpu_sc as plsc

N = 10000
E = 320000
D_IN = 128
HID = 16
D_OUT = 128

NC = 2
NS = 16
NW = NC * NS
Q = E // NW
CH = 80
NCHP = Q // CH
NP = 10240
RPT = NP // NS

_mesh = plsc.VectorSubcoreMesh(core_axis_name="c", subcore_axis_name="s")


def _deg_body(dst_hbm, ones_hbm, zero_hbm, out_hbm, dst_v, ones_v, accum_sh):
    cid = lax.axis_index("c")
    sid = lax.axis_index("s")
    wid = sid * NC + cid

    @pl.when(sid == 0)
    def _():
        pltpu.sync_copy(zero_hbm, accum_sh)

    plsc.subcore_barrier()
    pltpu.sync_copy(dst_hbm.at[wid], dst_v)
    pltpu.sync_copy(ones_hbm, ones_v)

    def step(j, carry):
        pltpu.sync_copy(ones_v, accum_sh.at[dst_v.at[j]], add=True)
        return carry

    lax.fori_loop(0, NCHP, step, 0)
    plsc.subcore_barrier()
    pltpu.sync_copy(accum_sh.at[pl.ds(sid * RPT, RPT)],
                    out_hbm.at[cid, pl.ds(sid * RPT, RPT)])


def _rsqrt16(d):
    y = plsc.bitcast(jnp.int32(0x5F3759DF) - (plsc.bitcast(d, jnp.int32) >> 1),
                     jnp.float32)
    hd = 0.5 * d
    y = y * (1.5 - hd * y * y)
    y = y * (1.5 - hd * y * y)
    return y


def _agg1_body(degp_hbm, h1_hbm, zero_hbm, src_hbm, dst_hbm,
               p_hbm, g1_hbm, dinv_hbm,
               src_v, dst_v, rows0, rows1, buf_a, buf_b, buf_g, buf_d,
               g_sh, accum_sh, sem0, sem1):
    cid = lax.axis_index("c")
    sid = lax.axis_index("s")
    wid = sid * NC + cid
    sl = pl.ds(sid * RPT, RPT)

    pltpu.sync_copy(degp_hbm.at[0, sl], buf_a)
    pltpu.sync_copy(degp_hbm.at[1, sl], buf_b)
    pltpu.sync_copy(h1_hbm.at[sl], buf_g)

    def ewd(i, carry):
        c = pl.ds(i * 16, 16)
        buf_d[c] = _rsqrt16(buf_a[c] + buf_b[c] + 1.0)
        return carry

    lax.fori_loop(0, RPT // 16, ewd, 0, unroll=4)

    def ew(i, carry):
        d16 = buf_d[pl.ds(i * 16, 16)]
        base = i * 16
        for k in range(16):
            buf_g[base + k] = d16[k] * buf_g[base + k]
        return carry

    lax.fori_loop(0, RPT // 16, ew, 0)
    pltpu.sync_copy(buf_d, dinv_hbm.at[sl])
    pltpu.sync_copy(buf_g, g1_hbm.at[sl])
    pltpu.sync_copy(buf_g, g_sh.at[sl])

    @pl.when(sid == 0)
    def _():
        pltpu.sync_copy(zero_hbm, accum_sh)

    plsc.subcore_barrier()
    pltpu.sync_copy(src_hbm.at[wid], src_v)
    pltpu.sync_copy(dst_hbm.at[wid], dst_v)

    pltpu.make_async_copy(g_sh.at[src_v.at[0]], rows0, sem0).start()
    pltpu.make_async_copy(g_sh.at[src_v.at[1]], rows1, sem1).start()

    def pair(i, carry):
        j = 2 * i
        pltpu.make_async_copy(g_sh.at[src_v.at[j]], rows0, sem0).wait()
        pltpu.sync_copy(rows0, accum_sh.at[dst_v.at[j]], add=True)
        pltpu.make_async_copy(g_sh.at[src_v.at[j + 2]], rows0, sem0).start()
        pltpu.make_async_copy(g_sh.at[src_v.at[j + 1]], rows1, sem1).wait()
        pltpu.sync_copy(rows1, accum_sh.at[dst_v.at[j + 1]], add=True)
        pltpu.make_async_copy(g_sh.at[src_v.at[j + 3]], rows1, sem1).start()
        return carry

    lax.fori_loop(0, (NCHP - 3) // 2, pair, 0)
    pltpu.make_async_copy(g_sh.at[src_v.at[NCHP - 3]], rows0, sem0).wait()
    pltpu.sync_copy(rows0, accum_sh.at[dst_v.at[NCHP - 3]], add=True)
    pltpu.make_async_copy(g_sh.at[src_v.at[NCHP - 2]], rows1, sem1).wait()
    pltpu.sync_copy(rows1, accum_sh.at[dst_v.at[NCHP - 2]], add=True)
    pltpu.async_copy(g_sh.at[src_v.at[NCHP - 1]], rows0, sem0).wait()
    pltpu.sync_copy(rows0, accum_sh.at[dst_v.at[NCHP - 1]], add=True)
    plsc.subcore_barrier()
    pltpu.sync_copy(accum_sh.at[sl], p_hbm.at[cid, sl])


def _agg2_body(p_hbm, g1_hbm, dinv_hbm, b1_hbm, zero_hbm, src_hbm, dst_hbm,
               q_hbm, g2_hbm,
               src_v, dst_v, rows0, rows1, buf_a, buf_b, buf_g, buf_d,
               buf_bias, g_sh, accum_sh, sem0, sem1):
    cid = lax.axis_index("c")
    sid = lax.axis_index("s")
    wid = sid * NC + cid
    sl = pl.ds(sid * RPT, RPT)

    pltpu.sync_copy(p_hbm.at[0, sl], buf_a)
    pltpu.sync_copy(p_hbm.at[1, sl], buf_b)
    pltpu.sync_copy(g1_hbm.at[sl], buf_g)
    pltpu.sync_copy(dinv_hbm.at[sl], buf_d)
    pltpu.sync_copy(b1_hbm, buf_bias)
    bias = buf_bias[0]

    def ew(i, carry):
        d16 = buf_d[pl.ds(i * 16, 16)]
        base = i * 16
        for k in range(16):
            r = base + k
            d = d16[k]
            s = buf_a[r] + buf_b[r] + buf_g[r]
            o = jnp.maximum(d * s + bias, 0.0)
            buf_g[r] = d * o
        return carry

    lax.fori_loop(0, RPT // 16, ew, 0)
    pltpu.sync_copy(buf_g, g_sh.at[sl])
    pltpu.sync_copy(buf_g, g2_hbm.at[sl])

    @pl.when(sid == 0)
    def _():
        pltpu.sync_copy(zero_hbm, accum_sh)

    plsc.subcore_barrier()
    pltpu.sync_copy(src_hbm.at[wid], src_v)
    pltpu.sync_copy(dst_hbm.at[wid], dst_v)

    pltpu.make_async_copy(g_sh.at[src_v.at[0]], rows0, sem0).start()
    pltpu.make_async_copy(g_sh.at[src_v.at[1]], rows1, sem1).start()

    def pair(i, carry):
        j = 2 * i
        pltpu.make_async_copy(g_sh.at[src_v.at[j]], rows0, sem0).wait()
        pltpu.sync_copy(rows0, accum_sh.at[dst_v.at[j]], add=True)
        pltpu.make_async_copy(g_sh.at[src_v.at[j + 2]], rows0, sem0).start()
        pltpu.make_async_copy(g_sh.at[src_v.at[j + 1]], rows1, sem1).wait()
        pltpu.sync_copy(rows1, accum_sh.at[dst_v.at[j + 1]], add=True)
        pltpu.make_async_copy(g_sh.at[src_v.at[j + 3]], rows1, sem1).start()
        return carry

    lax.fori_loop(0, (NCHP - 3) // 2, pair, 0)
    pltpu.make_async_copy(g_sh.at[src_v.at[NCHP - 3]], rows0, sem0).wait()
    pltpu.sync_copy(rows0, accum_sh.at[dst_v.at[NCHP - 3]], add=True)
    pltpu.make_async_copy(g_sh.at[src_v.at[NCHP - 2]], rows1, sem1).wait()
    pltpu.sync_copy(rows1, accum_sh.at[dst_v.at[NCHP - 2]], add=True)
    pltpu.async_copy(g_sh.at[src_v.at[NCHP - 1]], rows0, sem0).wait()
    pltpu.sync_copy(rows0, accum_sh.at[dst_v.at[NCHP - 1]], add=True)
    plsc.subcore_barrier()
    pltpu.sync_copy(accum_sh.at[sl], q_hbm.at[cid, sl])


_sc_params = pltpu.CompilerParams(
    use_tc_tiling_on_sc=False, needs_layout_passes=False
)

_deg_call = pl.kernel(
    _deg_body,
    out_type=jax.ShapeDtypeStruct((NC, NP), jnp.float32),
    mesh=_mesh,
    compiler_params=_sc_params,
    scratch_types=[
        pltpu.VMEM((NCHP, CH), jnp.int32),
        pltpu.VMEM((CH,), jnp.float32),
        pltpu.VMEM_SHARED((NP,), jnp.float32),
    ],
)

_agg1_call = pl.kernel(
    _agg1_body,
    out_type=[
        jax.ShapeDtypeStruct((NC, NP, HID), jnp.float32),
        jax.ShapeDtypeStruct((NP, HID), jnp.float32),
        jax.ShapeDtypeStruct((NP,), jnp.float32),
    ],
    mesh=_mesh,
    compiler_params=_sc_params,
    scratch_types=[
        pltpu.VMEM((NCHP, CH), jnp.int32),
        pltpu.VMEM((NCHP, CH), jnp.int32),
        pltpu.VMEM((CH, HID), jnp.float32),
        pltpu.VMEM((CH, HID), jnp.float32),
        pltpu.VMEM((RPT,), jnp.float32),
        pltpu.VMEM((RPT,), jnp.float32),
        pltpu.VMEM((RPT, HID), jnp.float32),
        pltpu.VMEM((RPT,), jnp.float32),
        pltpu.VMEM_SHARED((NP, HID), jnp.float32),
        pltpu.VMEM_SHARED((NP, HID), jnp.float32),
        pltpu.SemaphoreType.DMA,
        pltpu.SemaphoreType.DMA,
    ],
)

_agg2_call = pl.kernel(
    _agg2_body,
    out_type=[
        jax.ShapeDtypeStruct((NC, NP, HID), jnp.float32),
        jax.ShapeDtypeStruct((NP, HID), jnp.float32),
    ],
    mesh=_mesh,
    compiler_params=_sc_params,
    scratch_types=[
        pltpu.VMEM((NCHP, CH), jnp.int32),
        pltpu.VMEM((NCHP, CH), jnp.int32),
        pltpu.VMEM((CH, HID), jnp.float32),
        pltpu.VMEM((CH, HID), jnp.float32),
        pltpu.VMEM((RPT, HID), jnp.float32),
        pltpu.VMEM((RPT, HID), jnp.float32),
        pltpu.VMEM((RPT, HID), jnp.float32),
        pltpu.VMEM((RPT,), jnp.float32),
        pltpu.VMEM((1, HID), jnp.float32),
        pltpu.VMEM_SHARED((NP, HID), jnp.float32),
        pltpu.VMEM_SHARED((NP, HID), jnp.float32),
        pltpu.SemaphoreType.DMA,
        pltpu.SemaphoreType.DMA,
    ],
)


def _mm1_body(x_ref, w1_ref, h1_ref):
    h1_ref[pl.ds(0, N), :] = jnp.dot(
        x_ref[...], w1_ref[...], preferred_element_type=jnp.float32
    )
    h1_ref[pl.ds(N, NP - N), :] = jnp.zeros((NP - N, HID), jnp.float32)


def _fin_body(q_ref, g2_ref, dinv_ref, w2_ref, b2_ref, out_ref):
    z = dinv_ref[pl.ds(0, N), :] * (
        q_ref[0, pl.ds(0, N), :] + q_ref[1, pl.ds(0, N), :]
        + g2_ref[pl.ds(0, N), :]
    )
    out_ref[...] = (
        jnp.dot(z, w2_ref[...], preferred_element_type=jnp.float32)
        + b2_ref[...]
    )


_mm1_call = pl.pallas_call(
    _mm1_body,
    out_shape=jax.ShapeDtypeStruct((NP, HID), jnp.float32),
)

_fin_call = pl.pallas_call(
    _fin_body,
    out_shape=jax.ShapeDtypeStruct((N, D_OUT), jnp.float32),
)


def kernel(x, edge_index, W1, b1, W2, b2):
    dst3 = lax.optimization_barrier(edge_index[1].reshape(NW, NCHP, CH))
    src3 = lax.optimization_barrier(edge_index[0].reshape(NW, NCHP, CH))

    ones1 = jnp.ones((CH,), jnp.float32)
    zeros1 = jnp.zeros((NP,), jnp.float32)
    zeros_np = jnp.zeros((NP, HID), jnp.float32)

    degp = _deg_call(dst3, ones1, zeros1)
    h1 = _mm1_call(x, W1)
    p, g1, dinv = _agg1_call(degp, h1, zeros_np, src3, dst3)
    q, g2 = _agg2_call(p, g1, dinv, b1.reshape(1, HID), zeros_np, src3, dst3)
    return _fin_call(q, g2, dinv.reshape(NP, 1), W2, b2.reshape(1, D_OUT))


# speedup vs baseline: 1.0103x; 1.0103x over previous
import jax
import jax.numpy as jnp
from jax import lax
from jax.experimental import pallas as pl
from jax.experimental.pallas import tpu as pltpu
from jax.experimental.pallas import tpu_sc as plsc

N = 10000
E = 320000
D_IN = 128
HID = 16
D_OUT = 128

NC = 2
NS = 16
NW = NC * NS
Q = E // NW
CH = 80
NCHP = Q // CH
NP = 10240
RPT = NP // NS

_mesh = plsc.VectorSubcoreMesh(core_axis_name="c", subcore_axis_name="s")


def _deg_body(dst_hbm, ones_hbm, zero_hbm, out_hbm, dst_v, ones_v, accum_sh):
    cid = lax.axis_index("c")
    sid = lax.axis_index("s")
    wid = sid * NC + cid

    @pl.when(sid == 0)
    def _():
        pltpu.sync_copy(zero_hbm, accum_sh)

    plsc.subcore_barrier()
    pltpu.sync_copy(dst_hbm.at[wid], dst_v)
    pltpu.sync_copy(ones_hbm, ones_v)

    def step(j, carry):
        pltpu.sync_copy(ones_v, accum_sh.at[dst_v.at[j]], add=True)
        return carry

    lax.fori_loop(0, NCHP, step, 0)
    plsc.subcore_barrier()
    pltpu.sync_copy(accum_sh.at[pl.ds(sid * RPT, RPT)],
                    out_hbm.at[cid, pl.ds(sid * RPT, RPT)])


def _rsqrt16(d):
    y = plsc.bitcast(jnp.int32(0x5F3759DF) - (plsc.bitcast(d, jnp.int32) >> 1),
                     jnp.float32)
    hd = 0.5 * d
    y = y * (1.5 - hd * y * y)
    y = y * (1.5 - hd * y * y)
    return y


def _agg1_body(degp_hbm, h1_hbm, zero_hbm, src_hbm, dst_hbm,
               p_hbm, g1_hbm, dinv_hbm,
               src_v, dst_v, rows0, rows1, buf_a, buf_b, buf_g, buf_d,
               g_sh, accum_sh, sem0, sem1):
    cid = lax.axis_index("c")
    sid = lax.axis_index("s")
    wid = sid * NC + cid
    sl = pl.ds(sid * RPT, RPT)

    pltpu.sync_copy(degp_hbm.at[0, sl], buf_a)
    pltpu.sync_copy(degp_hbm.at[1, sl], buf_b)
    pltpu.sync_copy(h1_hbm.at[sl], buf_g)

    def ewd(i, carry):
        c = pl.ds(i * 16, 16)
        buf_d[c] = _rsqrt16(buf_a[c] + buf_b[c] + 1.0)
        return carry

    lax.fori_loop(0, RPT // 16, ewd, 0, unroll=4)

    def ew(i, carry):
        d16 = buf_d[pl.ds(i * 16, 16)]
        base = i * 16
        for k in range(16):
            buf_g[base + k] = d16[k] * buf_g[base + k]
        return carry

    lax.fori_loop(0, RPT // 16, ew, 0)
    pltpu.sync_copy(buf_d, dinv_hbm.at[sl])
    pltpu.sync_copy(buf_g, g1_hbm.at[sl])
    pltpu.sync_copy(buf_g, g_sh.at[sl])

    @pl.when(sid == 0)
    def _():
        pltpu.sync_copy(zero_hbm, accum_sh)

    plsc.subcore_barrier()
    pltpu.sync_copy(src_hbm.at[wid], src_v)
    pltpu.sync_copy(dst_hbm.at[wid], dst_v)

    pltpu.make_async_copy(g_sh.at[src_v.at[0]], rows0, sem0).start()
    pltpu.make_async_copy(g_sh.at[src_v.at[1]], rows1, sem1).start()

    def pair(i, carry):
        j = 2 * i
        pltpu.make_async_copy(g_sh.at[src_v.at[j]], rows0, sem0).wait()
        pltpu.sync_copy(rows0, accum_sh.at[dst_v.at[j]], add=True)
        pltpu.make_async_copy(g_sh.at[src_v.at[j + 2]], rows0, sem0).start()
        pltpu.make_async_copy(g_sh.at[src_v.at[j + 1]], rows1, sem1).wait()
        pltpu.sync_copy(rows1, accum_sh.at[dst_v.at[j + 1]], add=True)
        pltpu.make_async_copy(g_sh.at[src_v.at[j + 3]], rows1, sem1).start()
        return carry

    lax.fori_loop(0, (NCHP - 3) // 2, pair, 0)
    pltpu.make_async_copy(g_sh.at[src_v.at[NCHP - 3]], rows0, sem0).wait()
    pltpu.sync_copy(rows0, accum_sh.at[dst_v.at[NCHP - 3]], add=True)
    pltpu.make_async_copy(g_sh.at[src_v.at[NCHP - 2]], rows1, sem1).wait()
    pltpu.sync_copy(rows1, accum_sh.at[dst_v.at[NCHP - 2]], add=True)
    pltpu.async_copy(g_sh.at[src_v.at[NCHP - 1]], rows0, sem0).wait()
    pltpu.sync_copy(rows0, accum_sh.at[dst_v.at[NCHP - 1]], add=True)
    plsc.subcore_barrier()
    pltpu.sync_copy(accum_sh.at[sl], p_hbm.at[cid, sl])


def _agg2_body(p_hbm, g1_hbm, dinv_hbm, b1_hbm, zero_hbm, src_hbm, dst_hbm,
               q_hbm, g2_hbm,
               src_v, dst_v, rows0, rows1, buf_a, buf_b, buf_g, buf_d,
               buf_bias, g_sh, accum_sh, sem0, sem1):
    cid = lax.axis_index("c")
    sid = lax.axis_index("s")
    wid = sid * NC + cid
    sl = pl.ds(sid * RPT, RPT)

    pltpu.sync_copy(p_hbm.at[0, sl], buf_a)
    pltpu.sync_copy(p_hbm.at[1, sl], buf_b)
    pltpu.sync_copy(g1_hbm.at[sl], buf_g)
    pltpu.sync_copy(dinv_hbm.at[sl], buf_d)
    pltpu.sync_copy(b1_hbm, buf_bias)
    bias = buf_bias[0]

    def ew(i, carry):
        d16 = buf_d[pl.ds(i * 16, 16)]
        base = i * 16
        for k in range(16):
            r = base + k
            d = d16[k]
            s = buf_a[r] + buf_b[r] + buf_g[r]
            o = jnp.maximum(d * s + bias, 0.0)
            buf_g[r] = d * o
        return carry

    lax.fori_loop(0, RPT // 16, ew, 0)
    pltpu.sync_copy(buf_g, g_sh.at[sl])
    pltpu.sync_copy(buf_g, g2_hbm.at[sl])

    @pl.when(sid == 0)
    def _():
        pltpu.sync_copy(zero_hbm, accum_sh)

    plsc.subcore_barrier()
    pltpu.sync_copy(src_hbm.at[wid], src_v)
    pltpu.sync_copy(dst_hbm.at[wid], dst_v)

    pltpu.make_async_copy(g_sh.at[src_v.at[0]], rows0, sem0).start()
    pltpu.make_async_copy(g_sh.at[src_v.at[1]], rows1, sem1).start()

    def pair(i, carry):
        j = 2 * i
        pltpu.make_async_copy(g_sh.at[src_v.at[j]], rows0, sem0).wait()
        pltpu.sync_copy(rows0, accum_sh.at[dst_v.at[j]], add=True)
        pltpu.make_async_copy(g_sh.at[src_v.at[j + 2]], rows0, sem0).start()
        pltpu.make_async_copy(g_sh.at[src_v.at[j + 1]], rows1, sem1).wait()
        pltpu.sync_copy(rows1, accum_sh.at[dst_v.at[j + 1]], add=True)
        pltpu.make_async_copy(g_sh.at[src_v.at[j + 3]], rows1, sem1).start()
        return carry

    lax.fori_loop(0, (NCHP - 3) // 2, pair, 0)
    pltpu.make_async_copy(g_sh.at[src_v.at[NCHP - 3]], rows0, sem0).wait()
    pltpu.sync_copy(rows0, accum_sh.at[dst_v.at[NCHP - 3]], add=True)
    pltpu.make_async_copy(g_sh.at[src_v.at[NCHP - 2]], rows1, sem1).wait()
    pltpu.sync_copy(rows1, accum_sh.at[dst_v.at[NCHP - 2]], add=True)
    pltpu.async_copy(g_sh.at[src_v.at[NCHP - 1]], rows0, sem0).wait()
    pltpu.sync_copy(rows0, accum_sh.at[dst_v.at[NCHP - 1]], add=True)
    plsc.subcore_barrier()
    pltpu.sync_copy(accum_sh.at[sl], q_hbm.at[cid, sl])


_sc_params = pltpu.CompilerParams(
    use_tc_tiling_on_sc=False, needs_layout_passes=False
)

_deg_call = pl.kernel(
    _deg_body,
    out_type=jax.ShapeDtypeStruct((NC, NP), jnp.float32),
    mesh=_mesh,
    compiler_params=_sc_params,
    scratch_types=[
        pltpu.VMEM((NCHP, CH), jnp.int32),
        pltpu.VMEM((CH,), jnp.float32),
        pltpu.VMEM_SHARED((NP,), jnp.float32),
    ],
)

_agg1_call = pl.kernel(
    _agg1_body,
    out_type=[
        jax.ShapeDtypeStruct((NC, NP, HID), jnp.float32),
        jax.ShapeDtypeStruct((NP, HID), jnp.float32),
        jax.ShapeDtypeStruct((NP,), jnp.float32),
    ],
    mesh=_mesh,
    compiler_params=_sc_params,
    scratch_types=[
        pltpu.VMEM((NCHP, CH), jnp.int32),
        pltpu.VMEM((NCHP, CH), jnp.int32),
        pltpu.VMEM((CH, HID), jnp.float32),
        pltpu.VMEM((CH, HID), jnp.float32),
        pltpu.VMEM((RPT,), jnp.float32),
        pltpu.VMEM((RPT,), jnp.float32),
        pltpu.VMEM((RPT, HID), jnp.float32),
        pltpu.VMEM((RPT,), jnp.float32),
        pltpu.VMEM_SHARED((NP, HID), jnp.float32),
        pltpu.VMEM_SHARED((NP, HID), jnp.float32),
        pltpu.SemaphoreType.DMA,
        pltpu.SemaphoreType.DMA,
    ],
)

_agg2_call = pl.kernel(
    _agg2_body,
    out_type=[
        jax.ShapeDtypeStruct((NC, NP, HID), jnp.float32),
        jax.ShapeDtypeStruct((NP, HID), jnp.float32),
    ],
    mesh=_mesh,
    compiler_params=_sc_params,
    scratch_types=[
        pltpu.VMEM((NCHP, CH), jnp.int32),
        pltpu.VMEM((NCHP, CH), jnp.int32),
        pltpu.VMEM((CH, HID), jnp.float32),
        pltpu.VMEM((CH, HID), jnp.float32),
        pltpu.VMEM((RPT, HID), jnp.float32),
        pltpu.VMEM((RPT, HID), jnp.float32),
        pltpu.VMEM((RPT, HID), jnp.float32),
        pltpu.VMEM((RPT,), jnp.float32),
        pltpu.VMEM((1, HID), jnp.float32),
        pltpu.VMEM_SHARED((NP, HID), jnp.float32),
        pltpu.VMEM_SHARED((NP, HID), jnp.float32),
        pltpu.SemaphoreType.DMA,
        pltpu.SemaphoreType.DMA,
    ],
)


def _mm1_body(x_ref, w1_ref, h1_ref):
    h1_ref[pl.ds(0, N), :] = jnp.dot(
        x_ref[...], w1_ref[...], preferred_element_type=jnp.float32
    )
    h1_ref[pl.ds(N, NP - N), :] = jnp.zeros((NP - N, HID), jnp.float32)


def _fin_body(q_ref, g2_ref, dinv_ref, w2_ref, b2_ref, out_ref):
    z = dinv_ref[pl.ds(0, N), :] * (
        q_ref[0, pl.ds(0, N), :] + q_ref[1, pl.ds(0, N), :]
        + g2_ref[pl.ds(0, N), :]
    )
    out_ref[...] = (
        jnp.dot(z, w2_ref[...], preferred_element_type=jnp.float32)
        + b2_ref[...]
    )


_mm1_call = pl.pallas_call(
    _mm1_body,
    out_shape=jax.ShapeDtypeStruct((NP, HID), jnp.float32),
)

_fin_call = pl.pallas_call(
    _fin_body,
    out_shape=jax.ShapeDtypeStruct((N, D_OUT), jnp.float32),
)


def kernel(x, edge_index, W1, b1, W2, b2):
    src3 = edge_index[0].reshape(NW, NCHP, CH)
    dst3 = edge_index[1].reshape(NW, NCHP, CH)

    ones1 = jnp.ones((CH,), jnp.float32)
    zeros1 = jnp.zeros((NP,), jnp.float32)
    zeros_np = jnp.zeros((NP, HID), jnp.float32)

    degp = _deg_call(dst3, ones1, zeros1)
    h1 = _mm1_call(x, W1)
    p, g1, dinv = _agg1_call(degp, h1, zeros_np, src3, dst3)
    q, g2 = _agg2_call(p, g1, dinv, b1.reshape(1, HID), zeros_np, src3, dst3)
    return _fin_call(q, g2, dinv.reshape(NP, 1), W2, b2.reshape(1, D_OUT))


# speedup vs baseline: 1.1031x; 1.0918x over previous
import jax
import jax.numpy as jnp
from jax import lax
from jax.experimental import pallas as pl
from jax.experimental.pallas import tpu as pltpu
from jax.experimental.pallas import tpu_sc as plsc

N = 10000
E = 320000
D_IN = 128
HID = 16
D_OUT = 128

NC = 2
NS = 16
NW = NC * NS
Q = E // NW
CH = 80
NCHP = Q // CH
NP = 10240
RPT = NP // NS

_mesh = plsc.VectorSubcoreMesh(core_axis_name="c", subcore_axis_name="s")


def _deg_body(ei_hbm, ones_hbm, zero_hbm, out_hbm, dst_v, ones_v, accum_sh):
    cid = lax.axis_index("c")
    sid = lax.axis_index("s")
    wid = sid * NC + cid

    @pl.when(sid == 0)
    def _():
        pltpu.sync_copy(zero_hbm, accum_sh)

    plsc.subcore_barrier()
    pltpu.sync_copy(ei_hbm.at[1, wid], dst_v)
    pltpu.sync_copy(ones_hbm, ones_v)

    def step(j, carry):
        pltpu.sync_copy(ones_v, accum_sh.at[dst_v.at[j]], add=True)
        return carry

    lax.fori_loop(0, NCHP, step, 0)
    plsc.subcore_barrier()
    pltpu.sync_copy(accum_sh.at[pl.ds(sid * RPT, RPT)],
                    out_hbm.at[cid, pl.ds(sid * RPT, RPT)])


def _rsqrt16(d):
    y = plsc.bitcast(jnp.int32(0x5F3759DF) - (plsc.bitcast(d, jnp.int32) >> 1),
                     jnp.float32)
    hd = 0.5 * d
    y = y * (1.5 - hd * y * y)
    y = y * (1.5 - hd * y * y)
    return y


def _agg1_body(degp_hbm, h1_hbm, zero_hbm, ei_hbm,
               p_hbm, g1_hbm, dinv_hbm,
               src_v, dst_v, rows0, rows1, buf_a, buf_b, buf_g, buf_d,
               g_sh, accum_sh, sem0, sem1):
    cid = lax.axis_index("c")
    sid = lax.axis_index("s")
    wid = sid * NC + cid
    sl = pl.ds(sid * RPT, RPT)

    pltpu.sync_copy(degp_hbm.at[0, sl], buf_a)
    pltpu.sync_copy(degp_hbm.at[1, sl], buf_b)
    pltpu.sync_copy(h1_hbm.at[sl], buf_g)

    def ewd(i, carry):
        c = pl.ds(i * 16, 16)
        buf_d[c] = _rsqrt16(buf_a[c] + buf_b[c] + 1.0)
        return carry

    lax.fori_loop(0, RPT // 16, ewd, 0, unroll=4)

    def ew(i, carry):
        d16 = buf_d[pl.ds(i * 16, 16)]
        base = i * 16
        for k in range(16):
            buf_g[base + k] = d16[k] * buf_g[base + k]
        return carry

    lax.fori_loop(0, RPT // 16, ew, 0)
    pltpu.sync_copy(buf_d, dinv_hbm.at[sl])
    pltpu.sync_copy(buf_g, g1_hbm.at[sl])
    pltpu.sync_copy(buf_g, g_sh.at[sl])

    @pl.when(sid == 0)
    def _():
        pltpu.sync_copy(zero_hbm, accum_sh)

    plsc.subcore_barrier()
    pltpu.sync_copy(ei_hbm.at[0, wid], src_v)
    pltpu.sync_copy(ei_hbm.at[1, wid], dst_v)

    pltpu.make_async_copy(g_sh.at[src_v.at[0]], rows0, sem0).start()
    pltpu.make_async_copy(g_sh.at[src_v.at[1]], rows1, sem1).start()

    def pair(i, carry):
        j = 2 * i
        pltpu.make_async_copy(g_sh.at[src_v.at[j]], rows0, sem0).wait()
        pltpu.sync_copy(rows0, accum_sh.at[dst_v.at[j]], add=True)
        pltpu.make_async_copy(g_sh.at[src_v.at[j + 2]], rows0, sem0).start()
        pltpu.make_async_copy(g_sh.at[src_v.at[j + 1]], rows1, sem1).wait()
        pltpu.sync_copy(rows1, accum_sh.at[dst_v.at[j + 1]], add=True)
        pltpu.make_async_copy(g_sh.at[src_v.at[j + 3]], rows1, sem1).start()
        return carry

    lax.fori_loop(0, (NCHP - 3) // 2, pair, 0)
    pltpu.make_async_copy(g_sh.at[src_v.at[NCHP - 3]], rows0, sem0).wait()
    pltpu.sync_copy(rows0, accum_sh.at[dst_v.at[NCHP - 3]], add=True)
    pltpu.make_async_copy(g_sh.at[src_v.at[NCHP - 2]], rows1, sem1).wait()
    pltpu.sync_copy(rows1, accum_sh.at[dst_v.at[NCHP - 2]], add=True)
    pltpu.async_copy(g_sh.at[src_v.at[NCHP - 1]], rows0, sem0).wait()
    pltpu.sync_copy(rows0, accum_sh.at[dst_v.at[NCHP - 1]], add=True)
    plsc.subcore_barrier()
    pltpu.sync_copy(accum_sh.at[sl], p_hbm.at[cid, sl])


def _agg2_body(p_hbm, g1_hbm, dinv_hbm, b1_hbm, zero_hbm, ei_hbm,
               q_hbm, g2_hbm,
               src_v, dst_v, rows0, rows1, buf_a, buf_b, buf_g, buf_d,
               buf_bias, g_sh, accum_sh, sem0, sem1):
    cid = lax.axis_index("c")
    sid = lax.axis_index("s")
    wid = sid * NC + cid
    sl = pl.ds(sid * RPT, RPT)

    pltpu.sync_copy(p_hbm.at[0, sl], buf_a)
    pltpu.sync_copy(p_hbm.at[1, sl], buf_b)
    pltpu.sync_copy(g1_hbm.at[sl], buf_g)
    pltpu.sync_copy(dinv_hbm.at[sl], buf_d)
    pltpu.sync_copy(b1_hbm, buf_bias)
    bias = buf_bias[0]

    def ew(i, carry):
        d16 = buf_d[pl.ds(i * 16, 16)]
        base = i * 16
        for k in range(16):
            r = base + k
            d = d16[k]
            s = buf_a[r] + buf_b[r] + buf_g[r]
            o = jnp.maximum(d * s + bias, 0.0)
            buf_g[r] = d * o
        return carry

    lax.fori_loop(0, RPT // 16, ew, 0)
    pltpu.sync_copy(buf_g, g_sh.at[sl])
    pltpu.sync_copy(buf_g, g2_hbm.at[sl])

    @pl.when(sid == 0)
    def _():
        pltpu.sync_copy(zero_hbm, accum_sh)

    plsc.subcore_barrier()
    pltpu.sync_copy(ei_hbm.at[0, wid], src_v)
    pltpu.sync_copy(ei_hbm.at[1, wid], dst_v)

    pltpu.make_async_copy(g_sh.at[src_v.at[0]], rows0, sem0).start()
    pltpu.make_async_copy(g_sh.at[src_v.at[1]], rows1, sem1).start()

    def pair(i, carry):
        j = 2 * i
        pltpu.make_async_copy(g_sh.at[src_v.at[j]], rows0, sem0).wait()
        pltpu.sync_copy(rows0, accum_sh.at[dst_v.at[j]], add=True)
        pltpu.make_async_copy(g_sh.at[src_v.at[j + 2]], rows0, sem0).start()
        pltpu.make_async_copy(g_sh.at[src_v.at[j + 1]], rows1, sem1).wait()
        pltpu.sync_copy(rows1, accum_sh.at[dst_v.at[j + 1]], add=True)
        pltpu.make_async_copy(g_sh.at[src_v.at[j + 3]], rows1, sem1).start()
        return carry

    lax.fori_loop(0, (NCHP - 3) // 2, pair, 0)
    pltpu.make_async_copy(g_sh.at[src_v.at[NCHP - 3]], rows0, sem0).wait()
    pltpu.sync_copy(rows0, accum_sh.at[dst_v.at[NCHP - 3]], add=True)
    pltpu.make_async_copy(g_sh.at[src_v.at[NCHP - 2]], rows1, sem1).wait()
    pltpu.sync_copy(rows1, accum_sh.at[dst_v.at[NCHP - 2]], add=True)
    pltpu.async_copy(g_sh.at[src_v.at[NCHP - 1]], rows0, sem0).wait()
    pltpu.sync_copy(rows0, accum_sh.at[dst_v.at[NCHP - 1]], add=True)
    plsc.subcore_barrier()
    pltpu.sync_copy(accum_sh.at[sl], q_hbm.at[cid, sl])


_sc_params = pltpu.CompilerParams(
    use_tc_tiling_on_sc=False, needs_layout_passes=False
)

_deg_call = pl.kernel(
    _deg_body,
    out_type=jax.ShapeDtypeStruct((NC, NP), jnp.float32),
    mesh=_mesh,
    compiler_params=_sc_params,
    scratch_types=[
        pltpu.VMEM((NCHP, CH), jnp.int32),
        pltpu.VMEM((CH,), jnp.float32),
        pltpu.VMEM_SHARED((NP,), jnp.float32),
    ],
)

_agg1_call = pl.kernel(
    _agg1_body,
    out_type=[
        jax.ShapeDtypeStruct((NC, NP, HID), jnp.float32),
        jax.ShapeDtypeStruct((NP, HID), jnp.float32),
        jax.ShapeDtypeStruct((NP,), jnp.float32),
    ],
    mesh=_mesh,
    compiler_params=_sc_params,
    scratch_types=[
        pltpu.VMEM((NCHP, CH), jnp.int32),
        pltpu.VMEM((NCHP, CH), jnp.int32),
        pltpu.VMEM((CH, HID), jnp.float32),
        pltpu.VMEM((CH, HID), jnp.float32),
        pltpu.VMEM((RPT,), jnp.float32),
        pltpu.VMEM((RPT,), jnp.float32),
        pltpu.VMEM((RPT, HID), jnp.float32),
        pltpu.VMEM((RPT,), jnp.float32),
        pltpu.VMEM_SHARED((NP, HID), jnp.float32),
        pltpu.VMEM_SHARED((NP, HID), jnp.float32),
        pltpu.SemaphoreType.DMA,
        pltpu.SemaphoreType.DMA,
    ],
)

_agg2_call = pl.kernel(
    _agg2_body,
    out_type=[
        jax.ShapeDtypeStruct((NC, NP, HID), jnp.float32),
        jax.ShapeDtypeStruct((NP, HID), jnp.float32),
    ],
    mesh=_mesh,
    compiler_params=_sc_params,
    scratch_types=[
        pltpu.VMEM((NCHP, CH), jnp.int32),
        pltpu.VMEM((NCHP, CH), jnp.int32),
        pltpu.VMEM((CH, HID), jnp.float32),
        pltpu.VMEM((CH, HID), jnp.float32),
        pltpu.VMEM((RPT, HID), jnp.float32),
        pltpu.VMEM((RPT, HID), jnp.float32),
        pltpu.VMEM((RPT, HID), jnp.float32),
        pltpu.VMEM((RPT,), jnp.float32),
        pltpu.VMEM((1, HID), jnp.float32),
        pltpu.VMEM_SHARED((NP, HID), jnp.float32),
        pltpu.VMEM_SHARED((NP, HID), jnp.float32),
        pltpu.SemaphoreType.DMA,
        pltpu.SemaphoreType.DMA,
    ],
)


def _mm1_body(x_ref, w1_ref, h1_ref):
    h1_ref[pl.ds(0, N), :] = jnp.dot(
        x_ref[...], w1_ref[...], preferred_element_type=jnp.float32
    )
    h1_ref[pl.ds(N, NP - N), :] = jnp.zeros((NP - N, HID), jnp.float32)


def _fin_body(q_ref, g2_ref, dinv_ref, w2_ref, b2_ref, out_ref):
    z = dinv_ref[pl.ds(0, N), :] * (
        q_ref[0, pl.ds(0, N), :] + q_ref[1, pl.ds(0, N), :]
        + g2_ref[pl.ds(0, N), :]
    )
    out_ref[...] = (
        jnp.dot(z, w2_ref[...], preferred_element_type=jnp.float32)
        + b2_ref[...]
    )


_mm1_call = pl.pallas_call(
    _mm1_body,
    out_shape=jax.ShapeDtypeStruct((NP, HID), jnp.float32),
)

_fin_call = pl.pallas_call(
    _fin_body,
    out_shape=jax.ShapeDtypeStruct((N, D_OUT), jnp.float32),
)


def kernel(x, edge_index, W1, b1, W2, b2):
    ei4 = edge_index.reshape(2, NW, NCHP, CH)

    ones1 = jnp.ones((CH,), jnp.float32)
    zeros1 = jnp.zeros((NP,), jnp.float32)
    zeros_np = jnp.zeros((NP, HID), jnp.float32)

    degp = _deg_call(ei4, ones1, zeros1)
    h1 = _mm1_call(x, W1)
    p, g1, dinv = _agg1_call(degp, h1, zeros_np, ei4)
    q, g2 = _agg2_call(p, g1, dinv, b1.reshape(1, HID), zeros_np, ei4)
    return _fin_call(q, g2, dinv.reshape(NP, 1), W2, b2.reshape(1, D_OUT))
